# baseline (speedup 1.0000x reference)
import functools

import jax
import jax.numpy as jnp
from jax import lax
from jax.experimental import pallas as pl
from jax.experimental.pallas import tpu as pltpu
from jax.experimental.pallas import tpu_sc as plsc

B = 16384
EMB = 64
NUM_ROWS = 1000000
BK = 8192
SPLIT = 61 * BK
NBLK = 62
LBUF = NBLK * BK
NC = 2
NS = 16
NW = NC * NS
BPW = B // NW
SUB = 128


def _relayout_body(ua, ub_, ma, mb_, uo, mo):
    uo[:, 0:EMB] = ua[...].T
    uo[:, EMB:2 * EMB] = ub_[...].T
    mo[:, 0:EMB] = ma[...].T
    mo[:, EMB:2 * EMB] = mb_[...].T


def _relayout(ut, mt):
    lo_spec = pl.BlockSpec((EMB, BK), lambda i: (0, i))
    hi_spec = pl.BlockSpec((EMB, BK), lambda i: (0, i + SPLIT // BK))
    out_spec = pl.BlockSpec((BK, 2 * EMB), lambda i: (i, 0))
    return pl.pallas_call(
        _relayout_body,
        grid=(NBLK,),
        in_specs=[lo_spec, hi_spec, lo_spec, hi_spec],
        out_specs=[out_spec, out_spec],
        out_shape=[
            jax.ShapeDtypeStruct((LBUF, 2 * EMB), jnp.float32),
            jax.ShapeDtypeStruct((LBUF, 2 * EMB), jnp.float32),
        ],
    )(ut, ut, mt, mt)


def _gather_body(uid_hbm, pid_hbm, nid_hbm, utab_hbm, mtab_hbm,
                 uout_hbm, pout_hbm, nout_hbm,
                 idx_v, pidx_v, pa_v, pb_v, out_v, sem):
    wid = lax.axis_index("s") * NC + lax.axis_index("c")
    base = wid * BPW
    ids = (uid_hbm, pid_hbm, nid_hbm)
    tabs = (utab_hbm, mtab_hbm, mtab_hbm)
    outs = (uout_hbm, pout_hbm, nout_hbm)
    bufs = (pa_v, pb_v)
    nsub = BPW // SUB
    for t in range(3):
        pltpu.sync_copy(ids[t].at[pl.ds(base, BPW)], idx_v)

        def build_body(q, _):
            v = idx_v[pl.ds(q * 16, 16)]
            hi = jnp.where(v >= SPLIT, jnp.int32(SPLIT), jnp.int32(0))
            pidx_v[q // (SUB // 16), pl.ds((q % (SUB // 16)) * 16, 16)] = v - hi
            return _

        lax.fori_loop(0, BPW // 16, build_body, 0)

        cps = [None, None]
        cps[0] = pltpu.async_copy(tabs[t].at[pidx_v.at[0]], pa_v, sem)
        for i in range(nsub):
            buf = bufs[i % 2]
            cps[i % 2].wait()
            if i + 1 < nsub:
                cps[(i + 1) % 2] = pltpu.async_copy(
                    tabs[t].at[pidx_v.at[i + 1]], bufs[(i + 1) % 2], sem)

            def sel_body(g, _, i=i, buf=buf):
                xv = idx_v[pl.ds(i * SUB + g * 16, 16)]
                for r in range(16):
                    cb = jnp.where(xv[r] >= SPLIT, jnp.int32(EMB), jnp.int32(0))
                    jo = i * SUB + g * 16 + r
                    jp = g * 16 + r
                    for k in range(EMB // 16):
                        out_v[jo, pl.ds(16 * k, 16)] = (
                            buf[jp, pl.ds(cb + 16 * k, 16)])
                return _

            lax.fori_loop(0, SUB // 16, sel_body, 0)
        pltpu.sync_copy(out_v, outs[t].at[pl.ds(base, BPW)])


_sc_gather = functools.partial(
    pl.kernel,
    mesh=plsc.VectorSubcoreMesh(core_axis_name="c", subcore_axis_name="s"),
    out_type=[
        jax.ShapeDtypeStruct((B, EMB), jnp.float32),
        jax.ShapeDtypeStruct((B, EMB), jnp.float32),
        jax.ShapeDtypeStruct((B, EMB), jnp.float32),
    ],
    scratch_types=[
        pltpu.VMEM((BPW,), jnp.int32),
        pltpu.VMEM((BPW // SUB, SUB), jnp.int32),
        pltpu.VMEM((SUB, 2 * EMB), jnp.float32),
        pltpu.VMEM((SUB, 2 * EMB), jnp.float32),
        pltpu.VMEM((BPW, EMB), jnp.float32),
        pltpu.SemaphoreType.DMA,
    ],
)(_gather_body)


BM = 2048


def _mlp_body(ue, pe, ne, uW1, ub1, uW2, ub2, mW1, mb1, mW2, mb2,
              uo, po, no):
    hu = jnp.maximum(
        jnp.dot(ue[...], uW1[...], preferred_element_type=jnp.float32) + ub1[...], 0.0)
    uo[...] = jnp.dot(hu, uW2[...], preferred_element_type=jnp.float32) + ub2[...]
    hp = jnp.maximum(
        jnp.dot(pe[...], mW1[...], preferred_element_type=jnp.float32) + mb1[...], 0.0)
    po[...] = jnp.dot(hp, mW2[...], preferred_element_type=jnp.float32) + mb2[...]
    hn = jnp.maximum(
        jnp.dot(ne[...], mW1[...], preferred_element_type=jnp.float32) + mb1[...], 0.0)
    no[...] = jnp.dot(hn, mW2[...], preferred_element_type=jnp.float32) + mb2[...]


def _mlp_towers(ue, pe, ne, uW1, ub1, uW2, ub2, mW1, mb1, mW2, mb2):
    emb_spec = pl.BlockSpec((BM, EMB), lambda i: (i, 0))
    w_spec = pl.BlockSpec((EMB, 64), lambda i: (0, 0))
    w2_spec = pl.BlockSpec((64, 32), lambda i: (0, 0))
    b1_spec = pl.BlockSpec((1, 64), lambda i: (0, 0))
    b2_spec = pl.BlockSpec((1, 32), lambda i: (0, 0))
    out_spec = pl.BlockSpec((BM, 32), lambda i: (i, 0))
    return pl.pallas_call(
        _mlp_body,
        grid=(B // BM,),
        in_specs=[emb_spec, emb_spec, emb_spec,
                  w_spec, b1_spec, w2_spec, b2_spec,
                  w_spec, b1_spec, w2_spec, b2_spec],
        out_specs=[out_spec, out_spec, out_spec],
        out_shape=[
            jax.ShapeDtypeStruct((B, 32), jnp.float32),
            jax.ShapeDtypeStruct((B, 32), jnp.float32),
            jax.ShapeDtypeStruct((B, 32), jnp.float32),
        ],
    )(ue, pe, ne, uW1, ub1.reshape(1, 64), uW2, ub2.reshape(1, 32),
      mW1, mb1.reshape(1, 64), mW2, mb2.reshape(1, 32))


def kernel(user_ids, pos_movie_ids, neg_movie_ids, user_table, movie_table,
           uW1, ub1, uW2, ub2, mW1, mb1, mW2, mb2):
    ulines, mlines = _relayout(user_table.T, movie_table.T)
    ue, pe, ne = _sc_gather(user_ids, pos_movie_ids, neg_movie_ids,
                            ulines, mlines)
    return tuple(_mlp_towers(ue, pe, ne, uW1, ub1, uW2, ub2,
                             mW1, mb1, mW2, mb2))


# speedup vs baseline: 1.0495x; 1.0495x over previous
import functools

import jax
import jax.numpy as jnp
from jax import lax
from jax.experimental import pallas as pl
from jax.experimental.pallas import tpu as pltpu
from jax.experimental.pallas import tpu_sc as plsc

B = 16384
EMB = 64
NUM_ROWS = 1000000
BK = 8192
SPLIT = 61 * BK
NBLK = 62
LBUF = NBLK * BK
NC = 2
NS = 16
NW = NC * NS
BPW = B // NW
SUB = 128


def _relayout_body(ua, ub_, ma, mb_, uo, mo):
    uo[:, 0:EMB] = ua[...].T
    uo[:, EMB:2 * EMB] = ub_[...].T
    mo[:, 0:EMB] = ma[...].T
    mo[:, EMB:2 * EMB] = mb_[...].T


def _relayout(ut, mt):
    lo_spec = pl.BlockSpec((EMB, BK), lambda i: (0, i))
    hi_spec = pl.BlockSpec((EMB, BK), lambda i: (0, i + SPLIT // BK))
    out_spec = pl.BlockSpec((BK, 2 * EMB), lambda i: (i, 0))
    return pl.pallas_call(
        _relayout_body,
        grid=(NBLK,),
        in_specs=[lo_spec, hi_spec, lo_spec, hi_spec],
        out_specs=[out_spec, out_spec],
        out_shape=[
            jax.ShapeDtypeStruct((LBUF, 2 * EMB), jnp.float32),
            jax.ShapeDtypeStruct((LBUF, 2 * EMB), jnp.float32),
        ],
    )(ut, ut, mt, mt)


def _gather_body(uid_hbm, pid_hbm, nid_hbm, utab_hbm, mtab_hbm,
                 uout_hbm, pout_hbm, nout_hbm,
                 idx_v, pidx_v, pa_v, pb_v, out_v, sem):
    wid = lax.axis_index("s") * NC + lax.axis_index("c")
    base = wid * BPW
    ids = (uid_hbm, pid_hbm, nid_hbm)
    tabs = (utab_hbm, mtab_hbm, mtab_hbm)
    outs = (uout_hbm, pout_hbm, nout_hbm)
    bufs = (pa_v, pb_v)
    nsub = BPW // SUB
    for t in range(3):
        pltpu.sync_copy(ids[t].at[pl.ds(base, BPW)], idx_v)

        def build_body(q, _):
            v = idx_v[pl.ds(q * 16, 16)]
            hi = jnp.where(v >= SPLIT, jnp.int32(SPLIT), jnp.int32(0))
            pidx_v[q // (SUB // 16), pl.ds((q % (SUB // 16)) * 16, 16)] = v - hi
            return _

        lax.fori_loop(0, BPW // 16, build_body, 0)

        cps = [None, None]
        cps[0] = pltpu.async_copy(tabs[t].at[pidx_v.at[0]], pa_v, sem)
        for i in range(nsub):
            buf = bufs[i % 2]
            cps[i % 2].wait()
            if i + 1 < nsub:
                cps[(i + 1) % 2] = pltpu.async_copy(
                    tabs[t].at[pidx_v.at[i + 1]], bufs[(i + 1) % 2], sem)

            def sel_body(g, _, i=i, buf=buf):
                xv = idx_v[pl.ds(i * SUB + g * 16, 16)]
                for r in range(16):
                    cb = jnp.where(xv[r] >= SPLIT, jnp.int32(EMB), jnp.int32(0))
                    jo = i * SUB + g * 16 + r
                    jp = g * 16 + r
                    for k in range(EMB // 16):
                        out_v[jo, pl.ds(16 * k, 16)] = (
                            buf[jp, pl.ds(cb + 16 * k, 16)])
                return _

            lax.fori_loop(0, SUB // 16, sel_body, 0)
        pltpu.sync_copy(out_v, outs[t].at[pl.ds(base, BPW)])


_sc_gather = functools.partial(
    pl.kernel,
    mesh=plsc.VectorSubcoreMesh(core_axis_name="c", subcore_axis_name="s"),
    out_type=[
        jax.ShapeDtypeStruct((B, EMB), jnp.float32),
        jax.ShapeDtypeStruct((B, EMB), jnp.float32),
        jax.ShapeDtypeStruct((B, EMB), jnp.float32),
    ],
    scratch_types=[
        pltpu.VMEM((BPW,), jnp.int32),
        pltpu.VMEM((BPW // SUB, SUB), jnp.int32),
        pltpu.VMEM((SUB, 2 * EMB), jnp.float32),
        pltpu.VMEM((SUB, 2 * EMB), jnp.float32),
        pltpu.VMEM((BPW, EMB), jnp.float32),
        pltpu.SemaphoreType.DMA,
    ],
)(_gather_body)


BM = 2048


def _mlp_body(ue, pe, ne, uW1, ub1, uW2, ub2, mW1, mb1, mW2, mb2,
              uo, po, no):
    hu = jnp.maximum(
        jnp.dot(ue[...], uW1[...], preferred_element_type=jnp.float32) + ub1[...], 0.0)
    uo[...] = (jnp.dot(hu, uW2[...], preferred_element_type=jnp.float32) + ub2[...]).T
    hp = jnp.maximum(
        jnp.dot(pe[...], mW1[...], preferred_element_type=jnp.float32) + mb1[...], 0.0)
    po[...] = (jnp.dot(hp, mW2[...], preferred_element_type=jnp.float32) + mb2[...]).T
    hn = jnp.maximum(
        jnp.dot(ne[...], mW1[...], preferred_element_type=jnp.float32) + mb1[...], 0.0)
    no[...] = (jnp.dot(hn, mW2[...], preferred_element_type=jnp.float32) + mb2[...]).T


def _mlp_towers(ue, pe, ne, uW1, ub1, uW2, ub2, mW1, mb1, mW2, mb2):
    emb_spec = pl.BlockSpec((BM, EMB), lambda i: (i, 0))
    w_spec = pl.BlockSpec((EMB, 64), lambda i: (0, 0))
    w2_spec = pl.BlockSpec((64, 32), lambda i: (0, 0))
    b1_spec = pl.BlockSpec((1, 64), lambda i: (0, 0))
    b2_spec = pl.BlockSpec((1, 32), lambda i: (0, 0))
    out_spec = pl.BlockSpec((32, BM), lambda i: (0, i))
    return pl.pallas_call(
        _mlp_body,
        grid=(B // BM,),
        in_specs=[emb_spec, emb_spec, emb_spec,
                  w_spec, b1_spec, w2_spec, b2_spec,
                  w_spec, b1_spec, w2_spec, b2_spec],
        out_specs=[out_spec, out_spec, out_spec],
        out_shape=[
            jax.ShapeDtypeStruct((32, B), jnp.float32),
            jax.ShapeDtypeStruct((32, B), jnp.float32),
            jax.ShapeDtypeStruct((32, B), jnp.float32),
        ],
    )(ue, pe, ne, uW1, ub1.reshape(1, 64), uW2, ub2.reshape(1, 32),
      mW1, mb1.reshape(1, 64), mW2, mb2.reshape(1, 32))


def kernel(user_ids, pos_movie_ids, neg_movie_ids, user_table, movie_table,
           uW1, ub1, uW2, ub2, mW1, mb1, mW2, mb2):
    ulines, mlines = _relayout(user_table.T, movie_table.T)
    ue, pe, ne = _sc_gather(user_ids, pos_movie_ids, neg_movie_ids,
                            ulines, mlines)
    uo, po, no = _mlp_towers(ue, pe, ne, uW1, ub1, uW2, ub2,
                             mW1, mb1, mW2, mb2)
    return (uo.T, po.T, no.T)
